# baseline (speedup 1.0000x reference)
import functools

import jax
import jax.numpy as jnp
from jax import lax
from jax.experimental import pallas as pl
from jax.experimental.pallas import tpu as pltpu
from jax.experimental.pallas import tpu_sc as plsc

_TOPK = 2
_TM = 256
_SCATTER_CHUNK = 32
_COMBINE_CHUNK = 32
_NW = 32


def _gate_body(x_ref, gw_ref, gb_ref, sel_ref, wts_ref):
    x = x_ref[...]
    logits = jnp.dot(x, gw_ref[...], preferred_element_type=jnp.float32)
    logits = logits + gb_ref[...]
    n, e = logits.shape
    iota = lax.broadcasted_iota(jnp.int32, (n, e), 1)
    m1 = jnp.max(logits, axis=1, keepdims=True)
    i1 = jnp.min(jnp.where(logits == m1, iota, e), axis=1, keepdims=True)
    masked = jnp.where(iota == i1, -jnp.inf, logits)
    m2 = jnp.max(masked, axis=1, keepdims=True)
    i2 = jnp.min(jnp.where(masked == m2, iota, e), axis=1, keepdims=True)
    ex = jnp.exp(m2 - m1)
    w1 = 1.0 / (1.0 + ex)
    w2 = ex * w1
    sel_ref[...] = jnp.concatenate([i1, i2], axis=1)
    wts_ref[...] = jnp.concatenate([w1, w2], axis=1)


def _gate(x2d, gate_w, gate_b):
    n, dim = x2d.shape
    e = gate_w.shape[1]
    tm = min(n, 1024)
    sel, wts = pl.pallas_call(
        _gate_body,
        grid=(n // tm,),
        in_specs=[
            pl.BlockSpec((tm, dim), lambda t: (t, 0)),
            pl.BlockSpec((dim, e), lambda t: (0, 0)),
            pl.BlockSpec((1, e), lambda t: (0, 0)),
        ],
        out_specs=[
            pl.BlockSpec((tm, _TOPK), lambda t: (t, 0)),
            pl.BlockSpec((tm, _TOPK), lambda t: (t, 0)),
        ],
        out_shape=[
            jax.ShapeDtypeStruct((n, _TOPK), jnp.int32),
            jax.ShapeDtypeStruct((n, _TOPK), jnp.float32),
        ],
    )(x2d, gate_w, gate_b.reshape(1, e))
    return sel, wts


def _route(sel, n_experts, r_pad):
    e_flat = sel.reshape(-1)
    onehot = (e_flat[:, None] == jnp.arange(n_experts, dtype=jnp.int32)[None, :])
    oh = onehot.astype(jnp.int32)
    cum = jnp.cumsum(oh, axis=0)
    counts = cum[-1]
    rank = jnp.sum(cum * oh, axis=1) - 1
    padded = ((counts + _TM - 1) // _TM) * _TM
    starts = jnp.concatenate(
        [jnp.zeros((1,), jnp.int32), jnp.cumsum(padded)[:-1]])
    pos = starts[e_flat] + rank
    n_tiles = r_pad // _TM
    tile_starts = jnp.arange(n_tiles, dtype=jnp.int32) * _TM
    eot = jnp.clip(
        jnp.sum(tile_starts[:, None] >= starts[None, :], axis=1) - 1,
        0, n_experts - 1).astype(jnp.int32)
    used = (starts[-1] + padded[-1]).reshape(1).astype(jnp.int32)
    pos2 = pos.reshape(-1, _TOPK)
    return eot, used, pos2[:, 0], pos2[:, 1]


def _sc_scatter(x2d, pos0, pos1, r_pad):
    n, d = x2d.shape
    per_w = n // _NW
    chunk = _SCATTER_CHUNK
    n_chunks = per_w // chunk
    p0_3 = pos0.reshape(_NW, n_chunks, chunk)
    p1_3 = pos1.reshape(_NW, n_chunks, chunk)
    mesh = plsc.VectorSubcoreMesh(core_axis_name="c", subcore_axis_name="s", num_cores=2, num_subcores=16)

    @functools.partial(
        pl.kernel,
        out_type=jax.ShapeDtypeStruct((r_pad, d), jnp.float32),
        mesh=mesh,
        scratch_types=[
            pltpu.VMEM((n_chunks, chunk), jnp.int32),
            pltpu.VMEM((n_chunks, chunk), jnp.int32),
            pltpu.VMEM((chunk, d), jnp.float32),
            pltpu.VMEM((chunk, d), jnp.float32),
            pltpu.SemaphoreType.DMA,
            pltpu.SemaphoreType.DMA,
            pltpu.SemaphoreType.DMA,
            pltpu.SemaphoreType.DMA,
            pltpu.SemaphoreType.DMA,
            pltpu.SemaphoreType.DMA,
        ],
    )
    def k(x_hbm, p0_hbm, p1_hbm, xg_hbm, i0_v, i1_v, b0, b1,
          sl0, sl1, s0a, s0b, s1a, s1b):
        wid = lax.axis_index("s") * 2 + lax.axis_index("c")
        base_w = wid * per_w
        pltpu.sync_copy(p0_hbm.at[wid], i0_v)
        pltpu.sync_copy(p1_hbm.at[wid], i1_v)
        bufs = (b0, b1)
        sls = (sl0, sl1)
        ssa = (s0a, s1a)
        ssb = (s0b, s1b)
        lcp = [None] * n_chunks
        wa = [None] * n_chunks
        wb = [None] * n_chunks
        for i in range(n_chunks):
            j = i % 2
            if i >= 2:
                wa[i - 2].wait()
                wb[i - 2].wait()
            lcp[i] = pltpu.async_copy(
                x_hbm.at[pl.ds(base_w + i * chunk, chunk)], bufs[j], sls[j])
            lcp[i].wait()
            wa[i] = pltpu.async_copy(bufs[j], xg_hbm.at[i0_v.at[i]], ssa[j])
            wb[i] = pltpu.async_copy(bufs[j], xg_hbm.at[i1_v.at[i]], ssb[j])
        if n_chunks >= 2:
            wa[-2].wait()
            wb[-2].wait()
        wa[-1].wait()
        wb[-1].wait()

    return k(x2d, p0_3, p1_3)


_INV_SQRT2 = 0.7071067811865476


def _grouped_body(eot_ref, used_ref, x_ref, w1_ref, b1_ref, w2_ref,
                  b2_ref, out_ref):
    t = pl.program_id(0)
    valid = t * _TM < used_ref[0]

    @pl.when(valid)
    def _():
        x = x_ref[...].astype(jnp.bfloat16)
        h = jnp.dot(x, w1_ref[0], preferred_element_type=jnp.float32)
        h = h + b1_ref[0]
        h = 0.5 * h * (1.0 + lax.erf(h * _INV_SQRT2))
        o = jnp.dot(h.astype(jnp.bfloat16), w2_ref[0],
                    preferred_element_type=jnp.float32)
        out_ref[...] = o + b2_ref[0]

    @pl.when(jnp.logical_not(valid))
    def _():
        out_ref[...] = jnp.zeros_like(out_ref)


def _grouped_mlp(xg, eot, used, w1, b1, w2, b2):
    r, dim = xg.shape
    e, _, hid = w1.shape
    n_tiles = r // _TM
    grid_spec = pltpu.PrefetchScalarGridSpec(
        num_scalar_prefetch=2,
        grid=(n_tiles,),
        in_specs=[
            pl.BlockSpec((_TM, dim), lambda t, eot, used: (t, 0)),
            pl.BlockSpec((1, dim, hid), lambda t, eot, used: (eot[t], 0, 0)),
            pl.BlockSpec((1, 1, hid), lambda t, eot, used: (eot[t], 0, 0)),
            pl.BlockSpec((1, hid, dim), lambda t, eot, used: (eot[t], 0, 0)),
            pl.BlockSpec((1, 1, dim), lambda t, eot, used: (eot[t], 0, 0)),
        ],
        out_specs=pl.BlockSpec((_TM, dim), lambda t, eot, used: (t, 0)),
    )
    out = pl.pallas_call(
        _grouped_body,
        grid_spec=grid_spec,
        out_shape=jax.ShapeDtypeStruct((r, dim), jnp.float32),
        compiler_params=pltpu.CompilerParams(
            dimension_semantics=("arbitrary",),
        ),
    )(eot, used, xg, w1.astype(jnp.bfloat16), b1.reshape(e, 1, hid),
      w2.astype(jnp.bfloat16), b2.reshape(e, 1, dim))
    return out


def _sc_combine(rows, pos0, pos1, w0, w1, d):
    n = pos0.shape[0]
    per_w = n // _NW
    chunk = _COMBINE_CHUNK
    n_chunks = per_w // chunk
    p0_3 = pos0.reshape(_NW, n_chunks, chunk)
    p1_3 = pos1.reshape(_NW, n_chunks, chunk)
    w0_3 = w0.reshape(_NW, n_chunks, chunk)
    w1_3 = w1.reshape(_NW, n_chunks, chunk)
    mesh = plsc.VectorSubcoreMesh(core_axis_name="c", subcore_axis_name="s", num_cores=2, num_subcores=16)

    @functools.partial(
        pl.kernel,
        out_type=jax.ShapeDtypeStruct((n, d), jnp.float32),
        mesh=mesh,
        scratch_types=[
            pltpu.VMEM((n_chunks, chunk), jnp.int32),
            pltpu.VMEM((n_chunks, chunk), jnp.int32),
            pltpu.VMEM((n_chunks, chunk), jnp.float32),
            pltpu.VMEM((n_chunks, chunk), jnp.float32),
            pltpu.VMEM((chunk, d), jnp.float32),
            pltpu.VMEM((chunk, d), jnp.float32),
            pltpu.VMEM((chunk, d), jnp.float32),
            pltpu.VMEM((chunk, d), jnp.float32),
            pltpu.SemaphoreType.DMA,
            pltpu.SemaphoreType.DMA,
            pltpu.SemaphoreType.DMA,
            pltpu.SemaphoreType.DMA,
            pltpu.SemaphoreType.DMA,
            pltpu.SemaphoreType.DMA,
        ],
    )
    def k(rows_hbm, p0_hbm, p1_hbm, w0_hbm, w1_hbm, out_hbm,
          i0_v, i1_v, w0_v, w1_v, a0, a1, c0, c1,
          sa0, sa1, sc0, sc1, sw0, sw1):
        wid = lax.axis_index("s") * 2 + lax.axis_index("c")
        base_w = wid * per_w
        pltpu.sync_copy(p0_hbm.at[wid], i0_v)
        pltpu.sync_copy(p1_hbm.at[wid], i1_v)
        pltpu.sync_copy(w0_hbm.at[wid], w0_v)
        pltpu.sync_copy(w1_hbm.at[wid], w1_v)
        abufs = (a0, a1)
        cbufs = (c0, c1)
        sas = (sa0, sa1)
        scs = (sc0, sc1)
        sws = (sw0, sw1)
        ga = [None] * n_chunks
        gc = [None] * n_chunks
        wcp = [None] * n_chunks
        dnums = lax.GatherDimensionNumbers(
            offset_dims=(), collapsed_slice_dims=(0,), start_index_map=(0,))

        def _bcast(vec, lane_idx):
            idx = (jnp.zeros((16,), jnp.int32) + lane_idx)[:, None]
            return lax.gather(vec, idx, dimension_numbers=dnums,
                              slice_sizes=(1,),
                              mode=lax.GatherScatterMode.PROMISE_IN_BOUNDS)

        def add_chunk(j, i):
            for h in range(chunk // 16):
                w0_16 = w0_v[i, pl.ds(h * 16, 16)]
                w1_16 = w1_v[i, pl.ds(h * 16, 16)]

                def body(c16, cc):
                    w0b = _bcast(w0_16, c16)
                    w1b = _bcast(w1_16, c16)
                    row = h * 16 + c16
                    for dd in range(d // 16):
                        sl = pl.ds(dd * 16, 16)
                        abufs[j][row, sl] = (w0b * abufs[j][row, sl]
                                             + w1b * cbufs[j][row, sl])
                    return cc

                lax.fori_loop(0, 16, body, 0)

        for i in range(n_chunks):
            j = i % 2
            if i >= 2:
                wcp[i - 2].wait()
            ga[i] = pltpu.async_copy(rows_hbm.at[i0_v.at[i]], abufs[j], sas[j])
            gc[i] = pltpu.async_copy(rows_hbm.at[i1_v.at[i]], cbufs[j], scs[j])
            if i >= 1:
                jp = (i - 1) % 2
                ga[i - 1].wait()
                gc[i - 1].wait()
                add_chunk(jp, i - 1)
                wcp[i - 1] = pltpu.async_copy(
                    abufs[jp],
                    out_hbm.at[pl.ds(base_w + (i - 1) * chunk, chunk)],
                    sws[jp])
        jl = (n_chunks - 1) % 2
        ga[-1].wait()
        gc[-1].wait()
        add_chunk(jl, n_chunks - 1)
        wcp[-1] = pltpu.async_copy(
            abufs[jl],
            out_hbm.at[pl.ds(base_w + (n_chunks - 1) * chunk, chunk)],
            sws[jl])
        if n_chunks >= 2:
            wcp[-2].wait()
        wcp[-1].wait()

    return k(rows, p0_3, p1_3, w0_3, w1_3)


def kernel(x, gate_w, gate_b, w1, b1, w2, b2):
    b, s, dim = x.shape
    e = gate_w.shape[1]
    n = b * s
    r_pad = n * _TOPK + e * _TM
    x2d = x.reshape(n, dim)
    sel, wts = _gate(x2d, gate_w, gate_b)
    eot, used, pos0, pos1 = _route(sel, e, r_pad)
    xg = _sc_scatter(x2d, pos0, pos1, r_pad)
    rows = _grouped_mlp(xg, eot, used, w1, b1, w2, b2)
    out = _sc_combine(rows, pos0, pos1, wts[:, 0], wts[:, 1], dim)
    return out.reshape(b, s, dim), sel.reshape(b, s, _TOPK)


# speedup vs baseline: 1.1539x; 1.1539x over previous
import functools

import jax
import jax.numpy as jnp
from jax import lax
from jax.experimental import pallas as pl
from jax.experimental.pallas import tpu as pltpu
from jax.experimental.pallas import tpu_sc as plsc

_TOPK = 2
_TM = 128
_SCATTER_CHUNK = 32
_COMBINE_CHUNK = 32
_NW = 32


def _gate_body(x_ref, gw_ref, gb_ref, sel_ref, wts_ref, rank_ref, counts_ref,
               carry_ref):
    t = pl.program_id(0)

    @pl.when(t == 0)
    def _():
        carry_ref[...] = jnp.zeros_like(carry_ref)

    x = x_ref[...]
    logits = jnp.dot(x, gw_ref[...], preferred_element_type=jnp.float32)
    logits = logits + gb_ref[...]
    n, e = logits.shape
    iota = lax.broadcasted_iota(jnp.int32, (n, e), 1)
    m1 = jnp.max(logits, axis=1, keepdims=True)
    i1 = jnp.min(jnp.where(logits == m1, iota, e), axis=1, keepdims=True)
    masked = jnp.where(iota == i1, -jnp.inf, logits)
    m2 = jnp.max(masked, axis=1, keepdims=True)
    i2 = jnp.min(jnp.where(masked == m2, iota, e), axis=1, keepdims=True)
    ex = jnp.exp(m2 - m1)
    w1 = 1.0 / (1.0 + ex)
    w2 = ex * w1
    sel_ref[...] = jnp.concatenate([i1, i2], axis=1)
    wts_ref[...] = jnp.concatenate([w1, w2], axis=1)
    oh1 = (iota == i1).astype(jnp.float32)
    oh2 = (iota == i2).astype(jnp.float32)
    oh = oh1 + oh2
    ri = lax.broadcasted_iota(jnp.int32, (n, n), 0)
    ci = lax.broadcasted_iota(jnp.int32, (n, n), 1)
    ltri = (ri > ci).astype(jnp.float32)
    excl = jnp.dot(ltri, oh, preferred_element_type=jnp.float32)
    carry = carry_ref[...]
    base = excl + carry
    rank1 = jnp.sum(jnp.where(iota == i1, base, 0.0), axis=1, keepdims=True)
    rank2 = jnp.sum(jnp.where(iota == i2, base, 0.0), axis=1, keepdims=True)
    rank_ref[...] = jnp.concatenate([rank1, rank2], axis=1).astype(jnp.int32)
    carry_new = carry + jnp.sum(oh, axis=0, keepdims=True)
    carry_ref[...] = carry_new
    counts_ref[...] = carry_new.astype(jnp.int32)


def _gate(x2d, gate_w, gate_b):
    n, dim = x2d.shape
    e = gate_w.shape[1]
    tm = min(n, 1024)
    sel, wts, rank, counts = pl.pallas_call(
        _gate_body,
        grid=(n // tm,),
        in_specs=[
            pl.BlockSpec((tm, dim), lambda t: (t, 0)),
            pl.BlockSpec((dim, e), lambda t: (0, 0)),
            pl.BlockSpec((1, e), lambda t: (0, 0)),
        ],
        out_specs=[
            pl.BlockSpec((tm, _TOPK), lambda t: (t, 0)),
            pl.BlockSpec((tm, _TOPK), lambda t: (t, 0)),
            pl.BlockSpec((tm, _TOPK), lambda t: (t, 0)),
            pl.BlockSpec((1, e), lambda t: (0, 0)),
        ],
        out_shape=[
            jax.ShapeDtypeStruct((n, _TOPK), jnp.int32),
            jax.ShapeDtypeStruct((n, _TOPK), jnp.float32),
            jax.ShapeDtypeStruct((n, _TOPK), jnp.int32),
            jax.ShapeDtypeStruct((1, e), jnp.int32),
        ],
        scratch_shapes=[pltpu.VMEM((1, e), jnp.float32)],
        compiler_params=pltpu.CompilerParams(
            dimension_semantics=("arbitrary",),
        ),
    )(x2d, gate_w, gate_b.reshape(1, e))
    return sel, wts, rank, counts


def _route(sel, rank, counts, n_experts, r_pad):
    e_flat = sel.reshape(-1)
    counts = counts.reshape(-1)
    padded = ((counts + _TM - 1) // _TM) * _TM
    starts = jnp.concatenate(
        [jnp.zeros((1,), jnp.int32), jnp.cumsum(padded)[:-1]])
    erange = jnp.arange(n_experts, dtype=jnp.int32)
    start_of = jnp.sum(
        jnp.where(e_flat[:, None] == erange[None, :], starts[None, :], 0),
        axis=1)
    pos = start_of + rank.reshape(-1)
    n_tiles = r_pad // _TM
    tile_starts = jnp.arange(n_tiles, dtype=jnp.int32) * _TM
    eot = jnp.clip(
        jnp.sum(tile_starts[:, None] >= starts[None, :], axis=1) - 1,
        0, n_experts - 1).astype(jnp.int32)
    used = (starts[-1] + padded[-1]).reshape(1).astype(jnp.int32)
    pos2 = pos.reshape(-1, _TOPK)
    return eot, used, pos2[:, 0], pos2[:, 1]


def _sc_scatter(x2d, pos0, pos1, r_pad):
    n, d = x2d.shape
    per_w = n // _NW
    chunk = _SCATTER_CHUNK
    n_chunks = per_w // chunk
    p0_3 = pos0.reshape(_NW, n_chunks, chunk)
    p1_3 = pos1.reshape(_NW, n_chunks, chunk)
    mesh = plsc.VectorSubcoreMesh(core_axis_name="c", subcore_axis_name="s", num_cores=2, num_subcores=16)

    @functools.partial(
        pl.kernel,
        out_type=jax.ShapeDtypeStruct((r_pad, d), jnp.float32),
        mesh=mesh,
        scratch_types=[
            pltpu.VMEM((n_chunks, chunk), jnp.int32),
            pltpu.VMEM((n_chunks, chunk), jnp.int32),
            pltpu.VMEM((chunk, d), jnp.float32),
            pltpu.VMEM((chunk, d), jnp.float32),
            pltpu.SemaphoreType.DMA,
            pltpu.SemaphoreType.DMA,
            pltpu.SemaphoreType.DMA,
            pltpu.SemaphoreType.DMA,
            pltpu.SemaphoreType.DMA,
            pltpu.SemaphoreType.DMA,
        ],
    )
    def k(x_hbm, p0_hbm, p1_hbm, xg_hbm, i0_v, i1_v, b0, b1,
          sl0, sl1, s0a, s0b, s1a, s1b):
        wid = lax.axis_index("s") * 2 + lax.axis_index("c")
        base_w = wid * per_w
        pltpu.sync_copy(p0_hbm.at[wid], i0_v)
        pltpu.sync_copy(p1_hbm.at[wid], i1_v)
        bufs = (b0, b1)
        sls = (sl0, sl1)
        ssa = (s0a, s1a)
        ssb = (s0b, s1b)
        lcp = [None] * n_chunks
        wa = [None] * n_chunks
        wb = [None] * n_chunks
        for i in range(n_chunks):
            j = i % 2
            if i >= 2:
                wa[i - 2].wait()
                wb[i - 2].wait()
            lcp[i] = pltpu.async_copy(
                x_hbm.at[pl.ds(base_w + i * chunk, chunk)], bufs[j], sls[j])
            lcp[i].wait()
            wa[i] = pltpu.async_copy(bufs[j], xg_hbm.at[i0_v.at[i]], ssa[j])
            wb[i] = pltpu.async_copy(bufs[j], xg_hbm.at[i1_v.at[i]], ssb[j])
        if n_chunks >= 2:
            wa[-2].wait()
            wb[-2].wait()
        wa[-1].wait()
        wb[-1].wait()

    return k(x2d, p0_3, p1_3)


_INV_SQRT2 = 0.7071067811865476


def _grouped_body(eot_ref, used_ref, x_ref, w1_ref, b1_ref, w2_ref,
                  b2_ref, out_ref):
    t = pl.program_id(0)
    valid = t * _TM < used_ref[0]

    @pl.when(valid)
    def _():
        x = x_ref[...]
        h = jnp.dot(x, w1_ref[0], preferred_element_type=jnp.float32)
        h = h + b1_ref[0]
        h = 0.5 * h * (1.0 + lax.erf(h * _INV_SQRT2))
        o = jnp.dot(h, w2_ref[0], preferred_element_type=jnp.float32)
        out_ref[...] = o + b2_ref[0]

    @pl.when(jnp.logical_not(valid))
    def _():
        out_ref[...] = jnp.zeros_like(out_ref)


def _grouped_mlp(xg, eot, used, w1, b1, w2, b2):
    r, dim = xg.shape
    e, _, hid = w1.shape
    n_tiles = r // _TM
    grid_spec = pltpu.PrefetchScalarGridSpec(
        num_scalar_prefetch=2,
        grid=(n_tiles,),
        in_specs=[
            pl.BlockSpec((_TM, dim), lambda t, eot, used: (t, 0)),
            pl.BlockSpec((1, dim, hid), lambda t, eot, used: (eot[t], 0, 0)),
            pl.BlockSpec((1, 1, hid), lambda t, eot, used: (eot[t], 0, 0)),
            pl.BlockSpec((1, hid, dim), lambda t, eot, used: (eot[t], 0, 0)),
            pl.BlockSpec((1, 1, dim), lambda t, eot, used: (eot[t], 0, 0)),
        ],
        out_specs=pl.BlockSpec((_TM, dim), lambda t, eot, used: (t, 0)),
    )
    out = pl.pallas_call(
        _grouped_body,
        grid_spec=grid_spec,
        out_shape=jax.ShapeDtypeStruct((r, dim), jnp.float32),
        compiler_params=pltpu.CompilerParams(
            dimension_semantics=("arbitrary",),
        ),
    )(eot, used, xg, w1, b1.reshape(e, 1, hid), w2, b2.reshape(e, 1, dim))
    return out


def _sc_combine(rows, pos0, pos1, w0, w1, d):
    n = pos0.shape[0]
    per_w = n // _NW
    chunk = _COMBINE_CHUNK
    n_chunks = per_w // chunk
    p0_3 = pos0.reshape(_NW, n_chunks, chunk)
    p1_3 = pos1.reshape(_NW, n_chunks, chunk)
    w0_3 = w0.reshape(_NW, n_chunks, chunk)
    w1_3 = w1.reshape(_NW, n_chunks, chunk)
    mesh = plsc.VectorSubcoreMesh(core_axis_name="c", subcore_axis_name="s", num_cores=2, num_subcores=16)

    @functools.partial(
        pl.kernel,
        out_type=jax.ShapeDtypeStruct((n, d), jnp.float32),
        mesh=mesh,
        scratch_types=[
            pltpu.VMEM((n_chunks, chunk), jnp.int32),
            pltpu.VMEM((n_chunks, chunk), jnp.int32),
            pltpu.VMEM((n_chunks, chunk), jnp.float32),
            pltpu.VMEM((n_chunks, chunk), jnp.float32),
            pltpu.VMEM((chunk, d), jnp.float32),
            pltpu.VMEM((chunk, d), jnp.float32),
            pltpu.VMEM((chunk, d), jnp.float32),
            pltpu.VMEM((chunk, d), jnp.float32),
            pltpu.SemaphoreType.DMA,
            pltpu.SemaphoreType.DMA,
            pltpu.SemaphoreType.DMA,
            pltpu.SemaphoreType.DMA,
            pltpu.SemaphoreType.DMA,
            pltpu.SemaphoreType.DMA,
        ],
    )
    def k(rows_hbm, p0_hbm, p1_hbm, w0_hbm, w1_hbm, out_hbm,
          i0_v, i1_v, w0_v, w1_v, a0, a1, c0, c1,
          sa0, sa1, sc0, sc1, sw0, sw1):
        wid = lax.axis_index("s") * 2 + lax.axis_index("c")
        base_w = wid * per_w
        pltpu.sync_copy(p0_hbm.at[wid], i0_v)
        pltpu.sync_copy(p1_hbm.at[wid], i1_v)
        pltpu.sync_copy(w0_hbm.at[wid], w0_v)
        pltpu.sync_copy(w1_hbm.at[wid], w1_v)
        abufs = (a0, a1)
        cbufs = (c0, c1)
        sas = (sa0, sa1)
        scs = (sc0, sc1)
        sws = (sw0, sw1)
        ga = [None] * n_chunks
        gc = [None] * n_chunks
        wcp = [None] * n_chunks
        dnums = lax.GatherDimensionNumbers(
            offset_dims=(), collapsed_slice_dims=(0,), start_index_map=(0,))

        def _bcast(vec, lane_idx):
            idx = (jnp.zeros((16,), jnp.int32) + lane_idx)[:, None]
            return lax.gather(vec, idx, dimension_numbers=dnums,
                              slice_sizes=(1,),
                              mode=lax.GatherScatterMode.PROMISE_IN_BOUNDS)

        def add_chunk(j, i):
            for h in range(chunk // 16):
                w0_16 = w0_v[i, pl.ds(h * 16, 16)]
                w1_16 = w1_v[i, pl.ds(h * 16, 16)]

                def body(c16, cc):
                    w0b = _bcast(w0_16, c16)
                    w1b = _bcast(w1_16, c16)
                    row = h * 16 + c16
                    for dd in range(d // 16):
                        sl = pl.ds(dd * 16, 16)
                        abufs[j][row, sl] = (w0b * abufs[j][row, sl]
                                             + w1b * cbufs[j][row, sl])
                    return cc

                lax.fori_loop(0, 16, body, 0)

        for i in range(n_chunks):
            j = i % 2
            if i >= 2:
                wcp[i - 2].wait()
            ga[i] = pltpu.async_copy(rows_hbm.at[i0_v.at[i]], abufs[j], sas[j])
            gc[i] = pltpu.async_copy(rows_hbm.at[i1_v.at[i]], cbufs[j], scs[j])
            if i >= 1:
                jp = (i - 1) % 2
                ga[i - 1].wait()
                gc[i - 1].wait()
                add_chunk(jp, i - 1)
                wcp[i - 1] = pltpu.async_copy(
                    abufs[jp],
                    out_hbm.at[pl.ds(base_w + (i - 1) * chunk, chunk)],
                    sws[jp])
        jl = (n_chunks - 1) % 2
        ga[-1].wait()
        gc[-1].wait()
        add_chunk(jl, n_chunks - 1)
        wcp[-1] = pltpu.async_copy(
            abufs[jl],
            out_hbm.at[pl.ds(base_w + (n_chunks - 1) * chunk, chunk)],
            sws[jl])
        if n_chunks >= 2:
            wcp[-2].wait()
        wcp[-1].wait()

    return k(rows, p0_3, p1_3, w0_3, w1_3)


def kernel(x, gate_w, gate_b, w1, b1, w2, b2):
    b, s, dim = x.shape
    e = gate_w.shape[1]
    n = b * s
    r_pad = n * _TOPK + e * _TM
    x2d = x.reshape(n, dim)
    sel, wts, rank, counts = _gate(x2d, gate_w, gate_b)
    eot, used, pos0, pos1 = _route(sel, rank, counts, e, r_pad)
    xg = _sc_scatter(x2d, pos0, pos1, r_pad)
    rows = _grouped_mlp(xg, eot, used, w1, b1, w2, b2)
    out = _sc_combine(rows, pos0, pos1, wts[:, 0], wts[:, 1], dim)
    return out.reshape(b, s, dim), sel.reshape(b, s, _TOPK)


# speedup vs baseline: 1.2596x; 1.0917x over previous
import functools

import jax
import jax.numpy as jnp
from jax import lax
from jax.experimental import pallas as pl
from jax.experimental.pallas import tpu as pltpu
from jax.experimental.pallas import tpu_sc as plsc

_TOPK = 2
_TM = 256
_SCATTER_CHUNK = 32
_COMBINE_CHUNK = 32
_NW = 32


def _gate_body(x_ref, gw_ref, gb_ref, sel_ref, wts_ref, rank_ref, counts_ref,
               carry_ref):
    t = pl.program_id(0)

    @pl.when(t == 0)
    def _():
        carry_ref[...] = jnp.zeros_like(carry_ref)

    x = x_ref[...]
    logits = jnp.dot(x, gw_ref[...], preferred_element_type=jnp.float32)
    logits = logits + gb_ref[...]
    n, e = logits.shape
    iota = lax.broadcasted_iota(jnp.int32, (n, e), 1)
    m1 = jnp.max(logits, axis=1, keepdims=True)
    i1 = jnp.min(jnp.where(logits == m1, iota, e), axis=1, keepdims=True)
    masked = jnp.where(iota == i1, -jnp.inf, logits)
    m2 = jnp.max(masked, axis=1, keepdims=True)
    i2 = jnp.min(jnp.where(masked == m2, iota, e), axis=1, keepdims=True)
    ex = jnp.exp(m2 - m1)
    w1 = 1.0 / (1.0 + ex)
    w2 = ex * w1
    sel_ref[...] = jnp.concatenate([i1, i2], axis=1)
    wts_ref[...] = jnp.concatenate([w1, w2], axis=1)
    oh1 = (iota == i1).astype(jnp.float32)
    oh2 = (iota == i2).astype(jnp.float32)
    oh = oh1 + oh2
    ri = lax.broadcasted_iota(jnp.int32, (n, n), 0)
    ci = lax.broadcasted_iota(jnp.int32, (n, n), 1)
    ltri = (ri > ci).astype(jnp.float32)
    excl = jnp.dot(ltri, oh, preferred_element_type=jnp.float32)
    carry = carry_ref[...]
    base = excl + carry
    rank1 = jnp.sum(jnp.where(iota == i1, base, 0.0), axis=1, keepdims=True)
    rank2 = jnp.sum(jnp.where(iota == i2, base, 0.0), axis=1, keepdims=True)
    rank_ref[...] = jnp.concatenate([rank1, rank2], axis=1).astype(jnp.int32)
    carry_new = carry + jnp.sum(oh, axis=0, keepdims=True)
    carry_ref[...] = carry_new
    counts_ref[...] = carry_new.astype(jnp.int32)


def _gate(x2d, gate_w, gate_b):
    n, dim = x2d.shape
    e = gate_w.shape[1]
    tm = min(n, 1024)
    sel, wts, rank, counts = pl.pallas_call(
        _gate_body,
        grid=(n // tm,),
        in_specs=[
            pl.BlockSpec((tm, dim), lambda t: (t, 0)),
            pl.BlockSpec((dim, e), lambda t: (0, 0)),
            pl.BlockSpec((1, e), lambda t: (0, 0)),
        ],
        out_specs=[
            pl.BlockSpec((tm, _TOPK), lambda t: (t, 0)),
            pl.BlockSpec((tm, _TOPK), lambda t: (t, 0)),
            pl.BlockSpec((tm, _TOPK), lambda t: (t, 0)),
            pl.BlockSpec((1, e), lambda t: (0, 0)),
        ],
        out_shape=[
            jax.ShapeDtypeStruct((n, _TOPK), jnp.int32),
            jax.ShapeDtypeStruct((n, _TOPK), jnp.float32),
            jax.ShapeDtypeStruct((n, _TOPK), jnp.int32),
            jax.ShapeDtypeStruct((1, e), jnp.int32),
        ],
        scratch_shapes=[pltpu.VMEM((1, e), jnp.float32)],
        compiler_params=pltpu.CompilerParams(
            dimension_semantics=("arbitrary",),
        ),
    )(x2d, gate_w, gate_b.reshape(1, e))
    return sel, wts, rank, counts


def _route(sel, rank, counts, n_experts, r_pad):
    e_flat = sel.reshape(-1)
    counts = counts.reshape(-1)
    padded = ((counts + _TM - 1) // _TM) * _TM
    starts = jnp.concatenate(
        [jnp.zeros((1,), jnp.int32), jnp.cumsum(padded)[:-1]])
    erange = jnp.arange(n_experts, dtype=jnp.int32)
    start_of = jnp.sum(
        jnp.where(e_flat[:, None] == erange[None, :], starts[None, :], 0),
        axis=1)
    pos = start_of + rank.reshape(-1)
    n_tiles = r_pad // _TM
    tile_starts = jnp.arange(n_tiles, dtype=jnp.int32) * _TM
    eot = jnp.clip(
        jnp.sum(tile_starts[:, None] >= starts[None, :], axis=1) - 1,
        0, n_experts - 1).astype(jnp.int32)
    used = (starts[-1] + padded[-1]).reshape(1).astype(jnp.int32)
    pos2 = pos.reshape(-1, _TOPK)
    return eot, used, pos2[:, 0], pos2[:, 1]


def _sc_scatter(x2d, pos0, pos1, r_pad):
    n, d = x2d.shape
    per_w = n // _NW
    chunk = _SCATTER_CHUNK
    n_chunks = per_w // chunk
    p0_3 = pos0.reshape(_NW, n_chunks, chunk)
    p1_3 = pos1.reshape(_NW, n_chunks, chunk)
    mesh = plsc.VectorSubcoreMesh(core_axis_name="c", subcore_axis_name="s", num_cores=2, num_subcores=16)

    @functools.partial(
        pl.kernel,
        out_type=jax.ShapeDtypeStruct((r_pad, d), jnp.float32),
        mesh=mesh,
        scratch_types=[
            pltpu.VMEM((n_chunks, chunk), jnp.int32),
            pltpu.VMEM((n_chunks, chunk), jnp.int32),
            pltpu.VMEM((chunk, d), jnp.float32),
            pltpu.VMEM((chunk, d), jnp.float32),
            pltpu.SemaphoreType.DMA,
            pltpu.SemaphoreType.DMA,
            pltpu.SemaphoreType.DMA,
            pltpu.SemaphoreType.DMA,
            pltpu.SemaphoreType.DMA,
            pltpu.SemaphoreType.DMA,
        ],
    )
    def k(x_hbm, p0_hbm, p1_hbm, xg_hbm, i0_v, i1_v, b0, b1,
          sl0, sl1, s0a, s0b, s1a, s1b):
        wid = lax.axis_index("s") * 2 + lax.axis_index("c")
        base_w = wid * per_w
        pltpu.sync_copy(p0_hbm.at[wid], i0_v)
        pltpu.sync_copy(p1_hbm.at[wid], i1_v)
        bufs = (b0, b1)
        sls = (sl0, sl1)
        ssa = (s0a, s1a)
        ssb = (s0b, s1b)
        lcp = [None] * n_chunks
        wa = [None] * n_chunks
        wb = [None] * n_chunks
        for i in range(n_chunks):
            j = i % 2
            if i >= 2:
                wa[i - 2].wait()
                wb[i - 2].wait()
            lcp[i] = pltpu.async_copy(
                x_hbm.at[pl.ds(base_w + i * chunk, chunk)], bufs[j], sls[j])
            lcp[i].wait()
            wa[i] = pltpu.async_copy(bufs[j], xg_hbm.at[i0_v.at[i]], ssa[j])
            wb[i] = pltpu.async_copy(bufs[j], xg_hbm.at[i1_v.at[i]], ssb[j])
        if n_chunks >= 2:
            wa[-2].wait()
            wb[-2].wait()
        wa[-1].wait()
        wb[-1].wait()

    return k(x2d, p0_3, p1_3)


_INV_SQRT2 = 0.7071067811865476


def _grouped_body(eot_ref, used_ref, x_ref, w1_ref, b1_ref, w2_ref,
                  b2_ref, out_ref):
    t = pl.program_id(0)
    valid = t * _TM < used_ref[0]

    @pl.when(valid)
    def _():
        x = x_ref[...]
        h = jnp.dot(x, w1_ref[0], preferred_element_type=jnp.float32,
                    precision=lax.Precision.DEFAULT)
        h = h + b1_ref[0]
        h = 0.5 * h * (1.0 + lax.erf(h * _INV_SQRT2))
        o = jnp.dot(h, w2_ref[0], preferred_element_type=jnp.float32,
                    precision=lax.Precision.DEFAULT)
        out_ref[...] = o + b2_ref[0]

    @pl.when(jnp.logical_not(valid))
    def _():
        out_ref[...] = jnp.zeros_like(out_ref)


def _grouped_mlp(xg, eot, used, w1, b1, w2, b2):
    r, dim = xg.shape
    e, _, hid = w1.shape
    n_tiles = r // _TM
    grid_spec = pltpu.PrefetchScalarGridSpec(
        num_scalar_prefetch=2,
        grid=(n_tiles,),
        in_specs=[
            pl.BlockSpec((_TM, dim), lambda t, eot, used: (t, 0)),
            pl.BlockSpec((1, dim, hid), lambda t, eot, used: (eot[t], 0, 0)),
            pl.BlockSpec((1, 1, hid), lambda t, eot, used: (eot[t], 0, 0)),
            pl.BlockSpec((1, hid, dim), lambda t, eot, used: (eot[t], 0, 0)),
            pl.BlockSpec((1, 1, dim), lambda t, eot, used: (eot[t], 0, 0)),
        ],
        out_specs=pl.BlockSpec((_TM, dim), lambda t, eot, used: (t, 0)),
    )
    out = pl.pallas_call(
        _grouped_body,
        grid_spec=grid_spec,
        out_shape=jax.ShapeDtypeStruct((r, dim), jnp.float32),
        compiler_params=pltpu.CompilerParams(
            dimension_semantics=("arbitrary",),
        ),
    )(eot, used, xg, w1, b1.reshape(e, 1, hid), w2, b2.reshape(e, 1, dim))
    return out


def _sc_combine(rows, pos0, pos1, w0, w1, d):
    n = pos0.shape[0]
    per_w = n // _NW
    chunk = _COMBINE_CHUNK
    n_chunks = per_w // chunk
    p0_3 = pos0.reshape(_NW, n_chunks, chunk)
    p1_3 = pos1.reshape(_NW, n_chunks, chunk)
    w0_3 = w0.reshape(_NW, n_chunks, chunk)
    w1_3 = w1.reshape(_NW, n_chunks, chunk)
    mesh = plsc.VectorSubcoreMesh(core_axis_name="c", subcore_axis_name="s", num_cores=2, num_subcores=16)

    @functools.partial(
        pl.kernel,
        out_type=jax.ShapeDtypeStruct((n, d), jnp.float32),
        mesh=mesh,
        scratch_types=[
            pltpu.VMEM((n_chunks, chunk), jnp.int32),
            pltpu.VMEM((n_chunks, chunk), jnp.int32),
            pltpu.VMEM((n_chunks, chunk), jnp.float32),
            pltpu.VMEM((n_chunks, chunk), jnp.float32),
            pltpu.VMEM((chunk, d), jnp.float32),
            pltpu.VMEM((chunk, d), jnp.float32),
            pltpu.VMEM((chunk, d), jnp.float32),
            pltpu.VMEM((chunk, d), jnp.float32),
            pltpu.SemaphoreType.DMA,
            pltpu.SemaphoreType.DMA,
            pltpu.SemaphoreType.DMA,
            pltpu.SemaphoreType.DMA,
            pltpu.SemaphoreType.DMA,
            pltpu.SemaphoreType.DMA,
        ],
    )
    def k(rows_hbm, p0_hbm, p1_hbm, w0_hbm, w1_hbm, out_hbm,
          i0_v, i1_v, w0_v, w1_v, a0, a1, c0, c1,
          sa0, sa1, sc0, sc1, sw0, sw1):
        wid = lax.axis_index("s") * 2 + lax.axis_index("c")
        base_w = wid * per_w
        pltpu.sync_copy(p0_hbm.at[wid], i0_v)
        pltpu.sync_copy(p1_hbm.at[wid], i1_v)
        pltpu.sync_copy(w0_hbm.at[wid], w0_v)
        pltpu.sync_copy(w1_hbm.at[wid], w1_v)
        abufs = (a0, a1)
        cbufs = (c0, c1)
        sas = (sa0, sa1)
        scs = (sc0, sc1)
        sws = (sw0, sw1)
        ga = [None] * n_chunks
        gc = [None] * n_chunks
        wcp = [None] * n_chunks
        dnums = lax.GatherDimensionNumbers(
            offset_dims=(), collapsed_slice_dims=(0,), start_index_map=(0,))

        def _bcast(vec, lane_idx):
            idx = (jnp.zeros((16,), jnp.int32) + lane_idx)[:, None]
            return lax.gather(vec, idx, dimension_numbers=dnums,
                              slice_sizes=(1,),
                              mode=lax.GatherScatterMode.PROMISE_IN_BOUNDS)

        def add_chunk(j, i):
            for h in range(chunk // 16):
                w0_16 = w0_v[i, pl.ds(h * 16, 16)]
                w1_16 = w1_v[i, pl.ds(h * 16, 16)]

                def body(c16, cc):
                    w0b = _bcast(w0_16, c16)
                    w1b = _bcast(w1_16, c16)
                    row = h * 16 + c16
                    for dd in range(d // 16):
                        sl = pl.ds(dd * 16, 16)
                        abufs[j][row, sl] = (w0b * abufs[j][row, sl]
                                             + w1b * cbufs[j][row, sl])
                    return cc

                lax.fori_loop(0, 16, body, 0)

        for i in range(n_chunks):
            j = i % 2
            if i >= 2:
                wcp[i - 2].wait()
            ga[i] = pltpu.async_copy(rows_hbm.at[i0_v.at[i]], abufs[j], sas[j])
            gc[i] = pltpu.async_copy(rows_hbm.at[i1_v.at[i]], cbufs[j], scs[j])
            if i >= 1:
                jp = (i - 1) % 2
                ga[i - 1].wait()
                gc[i - 1].wait()
                add_chunk(jp, i - 1)
                wcp[i - 1] = pltpu.async_copy(
                    abufs[jp],
                    out_hbm.at[pl.ds(base_w + (i - 1) * chunk, chunk)],
                    sws[jp])
        jl = (n_chunks - 1) % 2
        ga[-1].wait()
        gc[-1].wait()
        add_chunk(jl, n_chunks - 1)
        wcp[-1] = pltpu.async_copy(
            abufs[jl],
            out_hbm.at[pl.ds(base_w + (n_chunks - 1) * chunk, chunk)],
            sws[jl])
        if n_chunks >= 2:
            wcp[-2].wait()
        wcp[-1].wait()

    return k(rows, p0_3, p1_3, w0_3, w1_3)


def kernel(x, gate_w, gate_b, w1, b1, w2, b2):
    b, s, dim = x.shape
    e = gate_w.shape[1]
    n = b * s
    r_pad = n * _TOPK + e * _TM
    x2d = x.reshape(n, dim)
    sel, wts, rank, counts = _gate(x2d, gate_w, gate_b)
    eot, used, pos0, pos1 = _route(sel, rank, counts, e, r_pad)
    xg = _sc_scatter(x2d, pos0, pos1, r_pad)
    rows = _grouped_mlp(xg, eot, used, w1, b1, w2, b2)
    out = _sc_combine(rows, pos0, pos1, wts[:, 0], wts[:, 1], dim)
    return out.reshape(b, s, dim), sel.reshape(b, s, _TOPK)


# speedup vs baseline: 1.2613x; 1.0013x over previous
import functools

import jax
import jax.numpy as jnp
from jax import lax
from jax.experimental import pallas as pl
from jax.experimental.pallas import tpu as pltpu
from jax.experimental.pallas import tpu_sc as plsc

_TOPK = 2
_TM = 256
_SCATTER_CHUNK = 32
_COMBINE_CHUNK = 32
_NW = 32


def _gate_body(x_ref, gw_ref, gb_ref, sel_ref, wts_ref, rank_ref, counts_ref,
               carry_ref):
    t = pl.program_id(0)

    @pl.when(t == 0)
    def _():
        carry_ref[...] = jnp.zeros_like(carry_ref)

    x = x_ref[...]
    logits = jnp.dot(x, gw_ref[...], preferred_element_type=jnp.float32)
    logits = logits + gb_ref[...]
    n, e = logits.shape
    iota = lax.broadcasted_iota(jnp.int32, (n, e), 1)
    m1 = jnp.max(logits, axis=1, keepdims=True)
    i1 = jnp.min(jnp.where(logits == m1, iota, e), axis=1, keepdims=True)
    masked = jnp.where(iota == i1, -jnp.inf, logits)
    m2 = jnp.max(masked, axis=1, keepdims=True)
    i2 = jnp.min(jnp.where(masked == m2, iota, e), axis=1, keepdims=True)
    ex = jnp.exp(m2 - m1)
    w1 = 1.0 / (1.0 + ex)
    w2 = ex * w1
    sel_ref[...] = jnp.concatenate([i1, i2], axis=1)
    wts_ref[...] = jnp.concatenate([w1, w2], axis=1)
    oh1 = (iota == i1).astype(jnp.float32)
    oh2 = (iota == i2).astype(jnp.float32)
    oh = oh1 + oh2
    ri = lax.broadcasted_iota(jnp.int32, (n, n), 0)
    ci = lax.broadcasted_iota(jnp.int32, (n, n), 1)
    ltri = (ri > ci).astype(jnp.float32)
    excl = jnp.dot(ltri, oh, preferred_element_type=jnp.float32)
    carry = carry_ref[...]
    base = excl + carry
    rank1 = jnp.sum(jnp.where(iota == i1, base, 0.0), axis=1, keepdims=True)
    rank2 = jnp.sum(jnp.where(iota == i2, base, 0.0), axis=1, keepdims=True)
    rank_ref[...] = jnp.concatenate([rank1, rank2], axis=1).astype(jnp.int32)
    carry_new = carry + jnp.sum(oh, axis=0, keepdims=True)
    carry_ref[...] = carry_new
    counts_ref[...] = carry_new.astype(jnp.int32)


def _gate(x2d, gate_w, gate_b):
    n, dim = x2d.shape
    e = gate_w.shape[1]
    tm = min(n, 512)
    sel, wts, rank, counts = pl.pallas_call(
        _gate_body,
        grid=(n // tm,),
        in_specs=[
            pl.BlockSpec((tm, dim), lambda t: (t, 0)),
            pl.BlockSpec((dim, e), lambda t: (0, 0)),
            pl.BlockSpec((1, e), lambda t: (0, 0)),
        ],
        out_specs=[
            pl.BlockSpec((tm, _TOPK), lambda t: (t, 0)),
            pl.BlockSpec((tm, _TOPK), lambda t: (t, 0)),
            pl.BlockSpec((tm, _TOPK), lambda t: (t, 0)),
            pl.BlockSpec((1, e), lambda t: (0, 0)),
        ],
        out_shape=[
            jax.ShapeDtypeStruct((n, _TOPK), jnp.int32),
            jax.ShapeDtypeStruct((n, _TOPK), jnp.float32),
            jax.ShapeDtypeStruct((n, _TOPK), jnp.int32),
            jax.ShapeDtypeStruct((1, e), jnp.int32),
        ],
        scratch_shapes=[pltpu.VMEM((1, e), jnp.float32)],
        compiler_params=pltpu.CompilerParams(
            dimension_semantics=("arbitrary",),
        ),
    )(x2d, gate_w, gate_b.reshape(1, e))
    return sel, wts, rank, counts


def _route(sel, rank, counts, n_experts, r_pad):
    e_flat = sel.reshape(-1)
    counts = counts.reshape(-1)
    padded = ((counts + _TM - 1) // _TM) * _TM
    starts = jnp.concatenate(
        [jnp.zeros((1,), jnp.int32), jnp.cumsum(padded)[:-1]])
    erange = jnp.arange(n_experts, dtype=jnp.int32)
    start_of = jnp.sum(
        jnp.where(e_flat[:, None] == erange[None, :], starts[None, :], 0),
        axis=1)
    pos = start_of + rank.reshape(-1)
    n_tiles = r_pad // _TM
    tile_starts = jnp.arange(n_tiles, dtype=jnp.int32) * _TM
    eot = jnp.clip(
        jnp.sum(tile_starts[:, None] >= starts[None, :], axis=1) - 1,
        0, n_experts - 1).astype(jnp.int32)
    used = (starts[-1] + padded[-1]).reshape(1).astype(jnp.int32)
    pos2 = pos.reshape(-1, _TOPK)
    return eot, used, pos2[:, 0], pos2[:, 1]


def _sc_scatter(x2d, pos0, pos1, r_pad):
    n, d = x2d.shape
    per_w = n // _NW
    chunk = _SCATTER_CHUNK
    n_chunks = per_w // chunk
    p0_3 = pos0.reshape(_NW, n_chunks, chunk)
    p1_3 = pos1.reshape(_NW, n_chunks, chunk)
    mesh = plsc.VectorSubcoreMesh(core_axis_name="c", subcore_axis_name="s", num_cores=2, num_subcores=16)

    @functools.partial(
        pl.kernel,
        out_type=jax.ShapeDtypeStruct((r_pad, d), jnp.float32),
        mesh=mesh,
        scratch_types=[
            pltpu.VMEM((n_chunks, chunk), jnp.int32),
            pltpu.VMEM((n_chunks, chunk), jnp.int32),
            pltpu.VMEM((chunk, d), jnp.float32),
            pltpu.VMEM((chunk, d), jnp.float32),
            pltpu.SemaphoreType.DMA,
            pltpu.SemaphoreType.DMA,
            pltpu.SemaphoreType.DMA,
            pltpu.SemaphoreType.DMA,
            pltpu.SemaphoreType.DMA,
            pltpu.SemaphoreType.DMA,
        ],
    )
    def k(x_hbm, p0_hbm, p1_hbm, xg_hbm, i0_v, i1_v, b0, b1,
          sl0, sl1, s0a, s0b, s1a, s1b):
        wid = lax.axis_index("s") * 2 + lax.axis_index("c")
        base_w = wid * per_w
        pltpu.sync_copy(p0_hbm.at[wid], i0_v)
        pltpu.sync_copy(p1_hbm.at[wid], i1_v)
        bufs = (b0, b1)
        sls = (sl0, sl1)
        ssa = (s0a, s1a)
        ssb = (s0b, s1b)
        lcp = [None] * n_chunks
        wa = [None] * n_chunks
        wb = [None] * n_chunks
        for i in range(n_chunks):
            j = i % 2
            if i >= 2:
                wa[i - 2].wait()
                wb[i - 2].wait()
            lcp[i] = pltpu.async_copy(
                x_hbm.at[pl.ds(base_w + i * chunk, chunk)], bufs[j], sls[j])
            lcp[i].wait()
            wa[i] = pltpu.async_copy(bufs[j], xg_hbm.at[i0_v.at[i]], ssa[j])
            wb[i] = pltpu.async_copy(bufs[j], xg_hbm.at[i1_v.at[i]], ssb[j])
        if n_chunks >= 2:
            wa[-2].wait()
            wb[-2].wait()
        wa[-1].wait()
        wb[-1].wait()

    return k(x2d, p0_3, p1_3)


_INV_SQRT2 = 0.7071067811865476


def _grouped_body(eot_ref, used_ref, x_ref, w1_ref, b1_ref, w2_ref,
                  b2_ref, out_ref):
    t = pl.program_id(0)
    valid = t * _TM < used_ref[0]

    @pl.when(valid)
    def _():
        x = x_ref[...]
        h = jnp.dot(x, w1_ref[0], preferred_element_type=jnp.float32)
        h = h + b1_ref[0]
        h = 0.5 * h * (1.0 + lax.erf(h * _INV_SQRT2))
        o = jnp.dot(h, w2_ref[0], preferred_element_type=jnp.float32)
        out_ref[...] = o + b2_ref[0]

    @pl.when(jnp.logical_not(valid))
    def _():
        out_ref[...] = jnp.zeros_like(out_ref)


def _grouped_mlp(xg, eot, used, w1, b1, w2, b2):
    r, dim = xg.shape
    e, _, hid = w1.shape
    n_tiles = r // _TM
    grid_spec = pltpu.PrefetchScalarGridSpec(
        num_scalar_prefetch=2,
        grid=(n_tiles,),
        in_specs=[
            pl.BlockSpec((_TM, dim), lambda t, eot, used: (t, 0)),
            pl.BlockSpec((1, dim, hid), lambda t, eot, used: (eot[t], 0, 0)),
            pl.BlockSpec((1, 1, hid), lambda t, eot, used: (eot[t], 0, 0)),
            pl.BlockSpec((1, hid, dim), lambda t, eot, used: (eot[t], 0, 0)),
            pl.BlockSpec((1, 1, dim), lambda t, eot, used: (eot[t], 0, 0)),
        ],
        out_specs=pl.BlockSpec((_TM, dim), lambda t, eot, used: (t, 0)),
    )
    out = pl.pallas_call(
        _grouped_body,
        grid_spec=grid_spec,
        out_shape=jax.ShapeDtypeStruct((r, dim), jnp.float32),
        compiler_params=pltpu.CompilerParams(
            dimension_semantics=("arbitrary",),
        ),
    )(eot, used, xg, w1, b1.reshape(e, 1, hid), w2, b2.reshape(e, 1, dim))
    return out


def _sc_combine(rows, pos0, pos1, w0, w1, d):
    n = pos0.shape[0]
    per_w = n // _NW
    chunk = _COMBINE_CHUNK
    n_chunks = per_w // chunk
    p0_3 = pos0.reshape(_NW, n_chunks, chunk)
    p1_3 = pos1.reshape(_NW, n_chunks, chunk)
    w0_3 = w0.reshape(_NW, n_chunks, chunk)
    w1_3 = w1.reshape(_NW, n_chunks, chunk)
    mesh = plsc.VectorSubcoreMesh(core_axis_name="c", subcore_axis_name="s", num_cores=2, num_subcores=16)

    @functools.partial(
        pl.kernel,
        out_type=jax.ShapeDtypeStruct((n, d), jnp.float32),
        mesh=mesh,
        scratch_types=[
            pltpu.VMEM((n_chunks, chunk), jnp.int32),
            pltpu.VMEM((n_chunks, chunk), jnp.int32),
            pltpu.VMEM((n_chunks, chunk), jnp.float32),
            pltpu.VMEM((n_chunks, chunk), jnp.float32),
            pltpu.VMEM((chunk, d), jnp.float32),
            pltpu.VMEM((chunk, d), jnp.float32),
            pltpu.VMEM((chunk, d), jnp.float32),
            pltpu.VMEM((chunk, d), jnp.float32),
            pltpu.SemaphoreType.DMA,
            pltpu.SemaphoreType.DMA,
            pltpu.SemaphoreType.DMA,
            pltpu.SemaphoreType.DMA,
            pltpu.SemaphoreType.DMA,
            pltpu.SemaphoreType.DMA,
        ],
    )
    def k(rows_hbm, p0_hbm, p1_hbm, w0_hbm, w1_hbm, out_hbm,
          i0_v, i1_v, w0_v, w1_v, a0, a1, c0, c1,
          sa0, sa1, sc0, sc1, sw0, sw1):
        wid = lax.axis_index("s") * 2 + lax.axis_index("c")
        base_w = wid * per_w
        pltpu.sync_copy(p0_hbm.at[wid], i0_v)
        pltpu.sync_copy(p1_hbm.at[wid], i1_v)
        pltpu.sync_copy(w0_hbm.at[wid], w0_v)
        pltpu.sync_copy(w1_hbm.at[wid], w1_v)
        abufs = (a0, a1)
        cbufs = (c0, c1)
        sas = (sa0, sa1)
        scs = (sc0, sc1)
        sws = (sw0, sw1)
        ga = [None] * n_chunks
        gc = [None] * n_chunks
        wcp = [None] * n_chunks
        dnums = lax.GatherDimensionNumbers(
            offset_dims=(), collapsed_slice_dims=(0,), start_index_map=(0,))

        def _bcast(vec, lane_idx):
            idx = (jnp.zeros((16,), jnp.int32) + lane_idx)[:, None]
            return lax.gather(vec, idx, dimension_numbers=dnums,
                              slice_sizes=(1,),
                              mode=lax.GatherScatterMode.PROMISE_IN_BOUNDS)

        def add_chunk(j, i):
            for h in range(chunk // 16):
                w0_16 = w0_v[i, pl.ds(h * 16, 16)]
                w1_16 = w1_v[i, pl.ds(h * 16, 16)]

                def body(c16, cc):
                    w0b = _bcast(w0_16, c16)
                    w1b = _bcast(w1_16, c16)
                    row = h * 16 + c16
                    for dd in range(d // 16):
                        sl = pl.ds(dd * 16, 16)
                        abufs[j][row, sl] = (w0b * abufs[j][row, sl]
                                             + w1b * cbufs[j][row, sl])
                    return cc

                lax.fori_loop(0, 16, body, 0)

        for i in range(n_chunks):
            j = i % 2
            if i >= 2:
                wcp[i - 2].wait()
            ga[i] = pltpu.async_copy(rows_hbm.at[i0_v.at[i]], abufs[j], sas[j])
            gc[i] = pltpu.async_copy(rows_hbm.at[i1_v.at[i]], cbufs[j], scs[j])
            if i >= 1:
                jp = (i - 1) % 2
                ga[i - 1].wait()
                gc[i - 1].wait()
                add_chunk(jp, i - 1)
                wcp[i - 1] = pltpu.async_copy(
                    abufs[jp],
                    out_hbm.at[pl.ds(base_w + (i - 1) * chunk, chunk)],
                    sws[jp])
        jl = (n_chunks - 1) % 2
        ga[-1].wait()
        gc[-1].wait()
        add_chunk(jl, n_chunks - 1)
        wcp[-1] = pltpu.async_copy(
            abufs[jl],
            out_hbm.at[pl.ds(base_w + (n_chunks - 1) * chunk, chunk)],
            sws[jl])
        if n_chunks >= 2:
            wcp[-2].wait()
        wcp[-1].wait()

    return k(rows, p0_3, p1_3, w0_3, w1_3)


def kernel(x, gate_w, gate_b, w1, b1, w2, b2):
    b, s, dim = x.shape
    e = gate_w.shape[1]
    n = b * s
    r_pad = n * _TOPK + e * _TM
    x2d = x.reshape(n, dim)
    sel, wts, rank, counts = _gate(x2d, gate_w, gate_b)
    eot, used, pos0, pos1 = _route(sel, rank, counts, e, r_pad)
    xg = _sc_scatter(x2d, pos0, pos1, r_pad)
    rows = _grouped_mlp(xg, eot, used, w1, b1, w2, b2)
    out = _sc_combine(rows, pos0, pos1, wts[:, 0], wts[:, 1], dim)
    return out.reshape(b, s, dim), sel.reshape(b, s, _TOPK)
